# baseline (speedup 1.0000x reference)
import functools

import jax
import jax.numpy as jnp
from jax import lax
from jax.experimental import pallas as pl
from jax.experimental.pallas import tpu as pltpu
from jax.experimental.pallas import tpu_sc as plsc


def _sc_gather(hist_pidx, user_pidx, item_pidx, utab2, itab2, n_hist, n_side):
    info = plsc.get_sparse_core_info()
    nc, ns = info.num_cores, info.num_subcores
    nw = nc * ns
    lanes = 128
    rows_w = n_hist // nw
    streams_w = rows_w // lanes
    s_per_chunk = 5
    chunks = streams_w // s_per_chunk
    chunk_rows = s_per_chunk * lanes
    side_w = n_side // nw // lanes

    mesh = plsc.VectorSubcoreMesh(core_axis_name="c", subcore_axis_name="s")
    f32 = jnp.float32

    @functools.partial(
        pl.kernel,
        out_type=(
            jax.ShapeDtypeStruct((n_hist, lanes), f32),
            jax.ShapeDtypeStruct((n_side, lanes), f32),
            jax.ShapeDtypeStruct((n_side, lanes), f32),
        ),
        mesh=mesh,
        compiler_params=pltpu.CompilerParams(use_tc_tiling_on_sc=True),
        scratch_types=[
            pltpu.VMEM((streams_w, lanes), jnp.int32),
            pltpu.VMEM((chunk_rows, lanes), f32),
            pltpu.VMEM((side_w, lanes), jnp.int32),
            pltpu.VMEM((lanes, lanes), f32),
            pltpu.SemaphoreType.DMA,
        ],
    )
    def gather_kernel(hist_idx_h, user_idx_h, item_idx_h, utab_h, itab_h,
                      hist_out, user_out, item_out,
                      idx_v, rows_v, sidx_v, srows_v, sem):
        wid = lax.axis_index("s") * nc + lax.axis_index("c")

        for tab, idx_h, out in ((utab_h, user_idx_h, user_out),
                                (itab_h, item_idx_h, item_out)):
            pltpu.sync_copy(idx_h.at[wid], sidx_v)
            for j in range(side_w):
                pltpu.async_copy(tab.at[sidx_v.at[j]], srows_v, sem).wait()
                pltpu.sync_copy(
                    srows_v,
                    out.at[pl.ds((wid * side_w + j) * lanes, lanes)])

        pltpu.sync_copy(hist_idx_h.at[wid], idx_v)

        @pl.loop(0, chunks)
        def _chunk(c):
            descs = [
                pltpu.async_copy(
                    itab_h.at[idx_v.at[c * s_per_chunk + jj]],
                    rows_v.at[pl.ds(jj * lanes, lanes)],
                    sem,
                )
                for jj in range(s_per_chunk)
            ]
            for desc in descs:
                desc.wait()
            pltpu.sync_copy(
                rows_v,
                hist_out.at[pl.ds(wid * rows_w + c * chunk_rows, chunk_rows)],
            )

    return gather_kernel(
        hist_pidx.reshape(nw, streams_w, lanes),
        user_pidx.reshape(nw, side_w, lanes),
        item_pidx.reshape(nw, side_w, lanes),
        utab2, itab2)


def _tc_dense(hist3, hpar3, user_pk, upar, item_pk, ipar,
              w_int, w1u_t, w1h_t, b1r, w2r, b2r, d):
    n_l, n_b, _ = hist3.shape
    f32 = jnp.float32

    def half(packed, par):
        return jnp.where(par == 1, packed[:, d:], packed[:, :d])

    def body(hist_ref, hpar_ref, u_ref, upar_ref, it_ref, ipar_ref,
             wint_ref, w1u_ref, w1h_ref, b1_ref, w2_ref, b2_ref,
             uout_ref, iout_ref, inter_ref, upart_s):
        step = pl.program_id(0)

        @pl.when(step == 0)
        def _init():
            u = half(u_ref[...], upar_ref[...])
            it = half(it_ref[...], ipar_ref[...])
            upart_s[...] = (
                jnp.dot(u, w1u_ref[...], preferred_element_type=f32)
                + b1_ref[...]
            )
            t = jnp.dot(u, wint_ref[...], preferred_element_type=f32)
            inter_ref[...] = jnp.sum(t * it, axis=1, keepdims=True)
            uout_ref[...] = u
            iout_ref[...] = it

        hist = half(hist_ref[0], hpar_ref[0])
        h = jnp.maximum(
            jnp.dot(hist, w1h_ref[...], preferred_element_type=f32)
            + upart_s[...],
            0.0,
        )
        a = jax.nn.sigmoid(
            jnp.sum(h * w2_ref[...], axis=1, keepdims=True) + b2_ref[0, 0]
        )
        uout_ref[...] += a * hist

    full = lambda shape: pl.BlockSpec(shape, lambda l: (0,) * len(shape))
    return pl.pallas_call(
        body,
        grid=(n_l,),
        in_specs=[
            pl.BlockSpec((1, n_b, 2 * d), lambda l: (l, 0, 0)),
            pl.BlockSpec((1, n_b, 1), lambda l: (l, 0, 0)),
            full((n_b, 2 * d)),
            full((n_b, 1)),
            full((n_b, 2 * d)),
            full((n_b, 1)),
            full((d, d)),
            full((d, d)),
            full((d, d)),
            full((1, d)),
            full((1, d)),
            full((1, 1)),
        ],
        out_specs=[full((n_b, d)), full((n_b, d)), full((n_b, 1))],
        out_shape=[
            jax.ShapeDtypeStruct((n_b, d), f32),
            jax.ShapeDtypeStruct((n_b, d), f32),
            jax.ShapeDtypeStruct((n_b, 1), f32),
        ],
        scratch_shapes=[pltpu.VMEM((n_b, d), f32)],
    )(hist3, hpar3, user_pk, upar, item_pk, ipar,
      w_int, w1u_t, w1h_t, b1r, w2r, b2r)


def kernel(user_ids, item_ids, user_history, user_table, item_table,
           W_int, w1, b1, w2, b2):
    n_b, n_l = user_history.shape
    d = user_table.shape[1]
    h_dim = w1.shape[0]

    utab2 = user_table.reshape(-1, 2 * d)
    itab2 = item_table.reshape(-1, 2 * d)

    hist_idx = user_history.T.reshape(-1)
    hist_rows, user_pk, item_pk = _sc_gather(
        hist_idx // 2, user_ids // 2, item_ids // 2,
        utab2, itab2, n_l * n_b, n_b)

    hpar3 = (user_history.T & 1).reshape(n_l, n_b, 1)
    user_out, item_emb, interaction = _tc_dense(
        hist_rows.reshape(n_l, n_b, 2 * d), hpar3,
        user_pk, (user_ids & 1).reshape(n_b, 1),
        item_pk, (item_ids & 1).reshape(n_b, 1),
        W_int, w1[:, :d].T, w1[:, d:].T,
        b1.reshape(1, h_dim), w2.reshape(1, h_dim),
        b2.reshape(1, 1).astype(jnp.float32), d)

    return (user_out, item_emb, interaction)


# speedup vs baseline: 1.4594x; 1.4594x over previous
import functools

import jax
import jax.numpy as jnp
from jax import lax
from jax.experimental import pallas as pl
from jax.experimental.pallas import tpu as pltpu
from jax.experimental.pallas import tpu_sc as plsc


def _tc_pack(table, n_cols):
    v, d = table.shape
    pairs = v // (2 * n_cols)
    off = pairs * n_cols
    tail_start = 2 * off
    steps = pairs + 1
    f32 = jnp.float32

    tail = jax.lax.slice(table, (tail_start, 0), (v, d))
    tail = jnp.pad(tail, ((0, n_cols - (v - tail_start)), (0, 0)))

    def body(lo_ref, hi_ref, tail_ref, out_ref):
        step = pl.program_id(0)

        @pl.when(step < pairs)
        def _pair():
            out_ref[...] = jnp.concatenate(
                [lo_ref[...].T, hi_ref[...].T], axis=1)

        @pl.when(step == pairs)
        def _tail():
            out_ref[...] = jnp.concatenate(
                [tail_ref[...], jnp.zeros_like(tail_ref[...])], axis=1)

    packed = pl.pallas_call(
        body,
        grid=(steps,),
        in_specs=[
            pl.BlockSpec((d, n_cols), lambda l: (0, jnp.minimum(l, pairs - 1))),
            pl.BlockSpec(
                (d, n_cols),
                lambda l: (0, jnp.minimum(l + pairs, 2 * pairs - 1))),
            pl.BlockSpec((n_cols, d), lambda l: (0, 0)),
        ],
        out_specs=pl.BlockSpec((n_cols, 2 * d), lambda l: (l, 0)),
        out_shape=jax.ShapeDtypeStruct((steps * n_cols, 2 * d), f32),
    )(table.T, table.T, tail)
    return packed, off, tail_start


def _sc_gather(hist_pidx, user_pidx, item_pidx, utab2, itab2, n_hist, n_side):
    info = plsc.get_sparse_core_info()
    nc, ns = info.num_cores, info.num_subcores
    nw = nc * ns
    lanes = 128
    rows_w = n_hist // nw
    streams_w = rows_w // lanes
    s_per_chunk = 5
    chunks = streams_w // s_per_chunk
    chunk_rows = s_per_chunk * lanes
    side_w = n_side // nw // lanes

    mesh = plsc.VectorSubcoreMesh(core_axis_name="c", subcore_axis_name="s")
    f32 = jnp.float32

    @functools.partial(
        pl.kernel,
        out_type=(
            jax.ShapeDtypeStruct((n_hist, lanes), f32),
            jax.ShapeDtypeStruct((n_side, lanes), f32),
            jax.ShapeDtypeStruct((n_side, lanes), f32),
        ),
        mesh=mesh,
        compiler_params=pltpu.CompilerParams(use_tc_tiling_on_sc=True),
        scratch_types=[
            pltpu.VMEM((streams_w, lanes), jnp.int32),
            pltpu.VMEM((chunk_rows, lanes), f32),
            pltpu.VMEM((side_w, lanes), jnp.int32),
            pltpu.VMEM((lanes, lanes), f32),
            pltpu.SemaphoreType.DMA,
        ],
    )
    def gather_kernel(hist_idx_h, user_idx_h, item_idx_h, utab_h, itab_h,
                      hist_out, user_out, item_out,
                      idx_v, rows_v, sidx_v, srows_v, sem):
        wid = lax.axis_index("s") * nc + lax.axis_index("c")

        for tab, idx_h, out in ((utab_h, user_idx_h, user_out),
                                (itab_h, item_idx_h, item_out)):
            pltpu.sync_copy(idx_h.at[wid], sidx_v)
            for j in range(side_w):
                pltpu.async_copy(tab.at[sidx_v.at[j]], srows_v, sem).wait()
                pltpu.sync_copy(
                    srows_v,
                    out.at[pl.ds((wid * side_w + j) * lanes, lanes)])

        pltpu.sync_copy(hist_idx_h.at[wid], idx_v)

        @pl.loop(0, chunks)
        def _chunk(c):
            descs = [
                pltpu.async_copy(
                    itab_h.at[idx_v.at[c * s_per_chunk + jj]],
                    rows_v.at[pl.ds(jj * lanes, lanes)],
                    sem,
                )
                for jj in range(s_per_chunk)
            ]
            for desc in descs:
                desc.wait()
            pltpu.sync_copy(
                rows_v,
                hist_out.at[pl.ds(wid * rows_w + c * chunk_rows, chunk_rows)],
            )

    return gather_kernel(
        hist_pidx.reshape(nw, streams_w, lanes),
        user_pidx.reshape(nw, side_w, lanes),
        item_pidx.reshape(nw, side_w, lanes),
        utab2, itab2)


def _tc_dense(hist3, hpar3, user_pk, upar, item_pk, ipar,
              w_int, w1u_t, w1h_t, b1r, w2r, b2r, d):
    n_l, n_b, _ = hist3.shape
    f32 = jnp.float32

    def half(packed, par):
        return jnp.where(par == 1, packed[:, d:], packed[:, :d])

    def body(hist_ref, hpar_ref, u_ref, upar_ref, it_ref, ipar_ref,
             wint_ref, w1u_ref, w1h_ref, b1_ref, w2_ref, b2_ref,
             uout_ref, iout_ref, inter_ref, upart_s):
        step = pl.program_id(0)

        @pl.when(step == 0)
        def _init():
            u = half(u_ref[...], upar_ref[...])
            it = half(it_ref[...], ipar_ref[...])
            upart_s[...] = (
                jnp.dot(u, w1u_ref[...], preferred_element_type=f32)
                + b1_ref[...]
            )
            t = jnp.dot(u, wint_ref[...], preferred_element_type=f32)
            inter_ref[...] = jnp.sum(t * it, axis=1, keepdims=True)
            uout_ref[...] = u
            iout_ref[...] = it

        hist = half(hist_ref[0], hpar_ref[0])
        h = jnp.maximum(
            jnp.dot(hist, w1h_ref[...], preferred_element_type=f32)
            + upart_s[...],
            0.0,
        )
        a = jax.nn.sigmoid(
            jnp.sum(h * w2_ref[...], axis=1, keepdims=True) + b2_ref[0, 0]
        )
        uout_ref[...] += a * hist

    full = lambda shape: pl.BlockSpec(shape, lambda l: (0,) * len(shape))
    return pl.pallas_call(
        body,
        grid=(n_l,),
        in_specs=[
            pl.BlockSpec((1, n_b, 2 * d), lambda l: (l, 0, 0)),
            pl.BlockSpec((1, n_b, 1), lambda l: (l, 0, 0)),
            full((n_b, 2 * d)),
            full((n_b, 1)),
            full((n_b, 2 * d)),
            full((n_b, 1)),
            full((d, d)),
            full((d, d)),
            full((d, d)),
            full((1, d)),
            full((1, d)),
            full((1, 1)),
        ],
        out_specs=[full((n_b, d)), full((n_b, d)), full((n_b, 1))],
        out_shape=[
            jax.ShapeDtypeStruct((n_b, d), f32),
            jax.ShapeDtypeStruct((n_b, d), f32),
            jax.ShapeDtypeStruct((n_b, 1), f32),
        ],
        scratch_shapes=[pltpu.VMEM((n_b, d), f32)],
    )(hist3, hpar3, user_pk, upar, item_pk, ipar,
      w_int, w1u_t, w1h_t, b1r, w2r, b2r)


def kernel(user_ids, item_ids, user_history, user_table, item_table,
           W_int, w1, b1, w2, b2):
    n_b, n_l = user_history.shape
    d = user_table.shape[1]
    h_dim = w1.shape[0]

    itab2, off, tail_start = _tc_pack(item_table, 2048)
    utab2, _, _ = _tc_pack(user_table, 2048)

    def map_ids(i):
        pidx = jnp.where(
            i >= tail_start, i - tail_start + off,
            jnp.where(i >= off, i - off, i))
        hbit = ((i >= off) & (i < tail_start)).astype(jnp.int32)
        return pidx, hbit

    hist_idx = user_history.T.reshape(-1)
    hpidx, hbit = map_ids(hist_idx)
    upidx, ubit = map_ids(user_ids)
    ipidx, ibit = map_ids(item_ids)
    hist_rows, user_pk, item_pk = _sc_gather(
        hpidx, upidx, ipidx, utab2, itab2, n_l * n_b, n_b)

    user_out, item_emb, interaction = _tc_dense(
        hist_rows.reshape(n_l, n_b, 2 * d), hbit.reshape(n_l, n_b, 1),
        user_pk, ubit.reshape(n_b, 1),
        item_pk, ibit.reshape(n_b, 1),
        W_int, w1[:, :d].T, w1[:, d:].T,
        b1.reshape(1, h_dim), w2.reshape(1, h_dim),
        b2.reshape(1, 1).astype(jnp.float32), d)

    return (user_out, item_emb, interaction)


# speedup vs baseline: 1.5243x; 1.0445x over previous
import functools

import jax
import jax.numpy as jnp
from jax import lax
from jax.experimental import pallas as pl
from jax.experimental.pallas import tpu as pltpu
from jax.experimental.pallas import tpu_sc as plsc


def _tc_pack(table, n_cols):
    v, d = table.shape
    pairs = v // (2 * n_cols)
    off = pairs * n_cols
    tail_start = 2 * off
    steps = pairs + 1
    f32 = jnp.float32

    tail = jax.lax.slice(table, (tail_start, 0), (v, d))
    tail = jnp.pad(tail, ((0, n_cols - (v - tail_start)), (0, 0)))

    def body(lo_ref, hi_ref, tail_ref, out_ref):
        step = pl.program_id(0)

        @pl.when(step < pairs)
        def _pair():
            out_ref[...] = jnp.concatenate(
                [lo_ref[...].T, hi_ref[...].T], axis=1)

        @pl.when(step == pairs)
        def _tail():
            out_ref[...] = jnp.concatenate(
                [tail_ref[...], jnp.zeros_like(tail_ref[...])], axis=1)

    packed = pl.pallas_call(
        body,
        grid=(steps,),
        in_specs=[
            pl.BlockSpec((d, n_cols), lambda l: (0, jnp.minimum(l, pairs - 1))),
            pl.BlockSpec(
                (d, n_cols),
                lambda l: (0, jnp.minimum(l + pairs, 2 * pairs - 1))),
            pl.BlockSpec((n_cols, d), lambda l: (0, 0)),
        ],
        out_specs=pl.BlockSpec((n_cols, 2 * d), lambda l: (l, 0)),
        out_shape=jax.ShapeDtypeStruct((steps * n_cols, 2 * d), f32),
    )(table.T, table.T, tail)
    return packed, off, tail_start


def _sc_info():
    info = plsc.get_sparse_core_info()
    return info.num_cores, info.num_subcores


def _sc_gather_hist(hist_pidx, item_pidx, itab2, n_hist, n_side):
    nc, ns = _sc_info()
    nw = nc * ns
    lanes = 128
    rows_w = n_hist // nw
    streams_w = rows_w // lanes
    s_per_chunk = 5
    chunks = streams_w // s_per_chunk
    chunk_rows = s_per_chunk * lanes
    side_w = n_side // nw // lanes

    mesh = plsc.VectorSubcoreMesh(core_axis_name="c", subcore_axis_name="s")
    f32 = jnp.float32

    @functools.partial(
        pl.kernel,
        out_type=(
            jax.ShapeDtypeStruct((n_hist, lanes), f32),
            jax.ShapeDtypeStruct((n_side, lanes), f32),
        ),
        mesh=mesh,
        compiler_params=pltpu.CompilerParams(use_tc_tiling_on_sc=True),
        scratch_types=[
            pltpu.VMEM((streams_w, lanes), jnp.int32),
            pltpu.VMEM((chunk_rows, lanes), f32),
            pltpu.VMEM((side_w, lanes), jnp.int32),
            pltpu.VMEM((lanes, lanes), f32),
            pltpu.SemaphoreType.DMA,
        ],
    )
    def gather_kernel(hist_idx_h, item_idx_h, itab_h,
                      hist_out, item_out,
                      idx_v, rows_v, sidx_v, srows_v, sem):
        wid = lax.axis_index("s") * nc + lax.axis_index("c")

        pltpu.sync_copy(item_idx_h.at[wid], sidx_v)
        for j in range(side_w):
            pltpu.async_copy(itab_h.at[sidx_v.at[j]], srows_v, sem).wait()
            pltpu.sync_copy(
                srows_v, item_out.at[pl.ds((wid * side_w + j) * lanes, lanes)])

        pltpu.sync_copy(hist_idx_h.at[wid], idx_v)

        @pl.loop(0, chunks)
        def _chunk(c):
            descs = [
                pltpu.async_copy(
                    itab_h.at[idx_v.at[c * s_per_chunk + jj]],
                    rows_v.at[pl.ds(jj * lanes, lanes)],
                    sem,
                )
                for jj in range(s_per_chunk)
            ]
            for desc in descs:
                desc.wait()
            pltpu.sync_copy(
                rows_v,
                hist_out.at[pl.ds(wid * rows_w + c * chunk_rows, chunk_rows)],
            )

    return gather_kernel(
        hist_pidx.reshape(nw, streams_w, lanes),
        item_pidx.reshape(nw, side_w, lanes),
        itab2)


def _sc_gather_user(user_pidx, utab2, n_side):
    nc, ns = _sc_info()
    nw = nc * ns
    lanes = 128
    side_w = n_side // nw // lanes

    mesh = plsc.VectorSubcoreMesh(core_axis_name="c", subcore_axis_name="s")

    @functools.partial(
        pl.kernel,
        out_type=jax.ShapeDtypeStruct((n_side, lanes), jnp.float32),
        mesh=mesh,
        compiler_params=pltpu.CompilerParams(use_tc_tiling_on_sc=True),
        scratch_types=[
            pltpu.VMEM((side_w, lanes), jnp.int32),
            pltpu.VMEM((lanes, lanes), jnp.float32),
            pltpu.SemaphoreType.DMA,
        ],
    )
    def gather_kernel(user_idx_h, utab_h, user_out, sidx_v, srows_v, sem):
        wid = lax.axis_index("s") * nc + lax.axis_index("c")
        pltpu.sync_copy(user_idx_h.at[wid], sidx_v)
        for j in range(side_w):
            pltpu.async_copy(utab_h.at[sidx_v.at[j]], srows_v, sem).wait()
            pltpu.sync_copy(
                srows_v, user_out.at[pl.ds((wid * side_w + j) * lanes, lanes)])

    return gather_kernel(user_pidx.reshape(nw, side_w, lanes), utab2)


def _tc_dense(hist3, hpar3, user_pk, upar, item_pk, ipar,
              w_int, w1u_t, w1h_t, b1r, w2r, b2r, d):
    n_l, n_b, _ = hist3.shape
    f32 = jnp.float32

    def half(packed, par):
        return jnp.where(par == 1, packed[:, d:], packed[:, :d])

    def body(hist_ref, hpar_ref, u_ref, upar_ref, it_ref, ipar_ref,
             wint_ref, w1u_ref, w1h_ref, b1_ref, w2_ref, b2_ref,
             uout_ref, iout_ref, inter_ref, upart_s):
        step = pl.program_id(0)

        @pl.when(step == 0)
        def _init():
            u = half(u_ref[...], upar_ref[...])
            it = half(it_ref[...], ipar_ref[...])
            upart_s[...] = (
                jnp.dot(u, w1u_ref[...], preferred_element_type=f32)
                + b1_ref[...]
            )
            t = jnp.dot(u, wint_ref[...], preferred_element_type=f32)
            inter_ref[...] = jnp.sum(t * it, axis=1, keepdims=True)
            uout_ref[...] = u
            iout_ref[...] = it

        cols = jax.lax.broadcasted_iota(jnp.int32, hpar_ref.shape, 1)
        hp = jnp.sum(
            jnp.where(cols == step, hpar_ref[...], 0), axis=1, keepdims=True)
        hist = half(hist_ref[0], hp)
        h = jnp.maximum(
            jnp.dot(hist, w1h_ref[...], preferred_element_type=f32)
            + upart_s[...],
            0.0,
        )
        a = jax.nn.sigmoid(
            jnp.sum(h * w2_ref[...], axis=1, keepdims=True) + b2_ref[0, 0]
        )
        uout_ref[...] += a * hist

    full = lambda shape: pl.BlockSpec(shape, lambda l: (0,) * len(shape))
    return pl.pallas_call(
        body,
        grid=(n_l,),
        in_specs=[
            pl.BlockSpec((1, n_b, 2 * d), lambda l: (l, 0, 0)),
            full((n_b, n_l)),
            full((n_b, 2 * d)),
            full((n_b, 1)),
            full((n_b, 2 * d)),
            full((n_b, 1)),
            full((d, d)),
            full((d, d)),
            full((d, d)),
            full((1, d)),
            full((1, d)),
            full((1, 1)),
        ],
        out_specs=[full((n_b, d)), full((n_b, d)), full((n_b, 1))],
        out_shape=[
            jax.ShapeDtypeStruct((n_b, d), f32),
            jax.ShapeDtypeStruct((n_b, d), f32),
            jax.ShapeDtypeStruct((n_b, 1), f32),
        ],
        scratch_shapes=[pltpu.VMEM((n_b, d), f32)],
    )(hist3, hpar3, user_pk, upar, item_pk, ipar,
      w_int, w1u_t, w1h_t, b1r, w2r, b2r)


def kernel(user_ids, item_ids, user_history, user_table, item_table,
           W_int, w1, b1, w2, b2):
    n_b, n_l = user_history.shape
    d = user_table.shape[1]
    h_dim = w1.shape[0]

    itab2, off, tail_start = _tc_pack(item_table, 2048)
    utab2, _, _ = _tc_pack(user_table, 2048)

    def map_ids(i):
        pidx = jnp.where(
            i >= tail_start, i - tail_start + off,
            jnp.where(i >= off, i - off, i))
        hbit = ((i >= off) & (i < tail_start)).astype(jnp.int32)
        return pidx, hbit

    hist_idx = user_history.T.reshape(-1)
    hpidx, _ = map_ids(hist_idx)
    upidx, ubit = map_ids(user_ids)
    ipidx, ibit = map_ids(item_ids)
    hbit_bm = ((user_history >= off)
               & (user_history < tail_start)).astype(jnp.int32)
    hist_rows, item_pk = _sc_gather_hist(
        hpidx, ipidx, itab2, n_l * n_b, n_b)
    user_pk = _sc_gather_user(upidx, utab2, n_b)

    user_out, item_emb, interaction = _tc_dense(
        hist_rows.reshape(n_l, n_b, 2 * d), hbit_bm,
        user_pk, ubit.reshape(n_b, 1),
        item_pk, ibit.reshape(n_b, 1),
        W_int, w1[:, :d].T, w1[:, d:].T,
        b1.reshape(1, h_dim), w2.reshape(1, h_dim),
        b2.reshape(1, 1).astype(jnp.float32), d)

    return (user_out, item_emb, interaction)


# speedup vs baseline: 1.8297x; 1.2003x over previous
import functools

import jax
import jax.numpy as jnp
from jax import lax
from jax.experimental import pallas as pl
from jax.experimental.pallas import tpu as pltpu
from jax.experimental.pallas import tpu_sc as plsc


def _tc_pack(table, n_cols):
    v, d = table.shape
    pairs = v // (2 * n_cols)
    off = pairs * n_cols
    tail_start = 2 * off
    steps = pairs + 1
    f32 = jnp.float32

    tail = jax.lax.slice(table, (tail_start, 0), (v, d))
    tail = jnp.pad(tail, ((0, n_cols - (v - tail_start)), (0, 0)))

    def body(lo_ref, hi_ref, tail_ref, out_ref):
        step = pl.program_id(0)

        @pl.when(step < pairs)
        def _pair():
            out_ref[...] = jnp.concatenate(
                [lo_ref[...].T, hi_ref[...].T], axis=1)

        @pl.when(step == pairs)
        def _tail():
            out_ref[...] = jnp.concatenate(
                [tail_ref[...], jnp.zeros_like(tail_ref[...])], axis=1)

    packed = pl.pallas_call(
        body,
        grid=(steps,),
        in_specs=[
            pl.BlockSpec((d, n_cols), lambda l: (0, jnp.minimum(l, pairs - 1))),
            pl.BlockSpec(
                (d, n_cols),
                lambda l: (0, jnp.minimum(l + pairs, 2 * pairs - 1))),
            pl.BlockSpec((n_cols, d), lambda l: (0, 0)),
        ],
        out_specs=pl.BlockSpec((n_cols, 2 * d), lambda l: (l, 0)),
        out_shape=jax.ShapeDtypeStruct((steps * n_cols, 2 * d), f32),
    )(table.T, table.T, tail)
    return packed, off, tail_start


def _sc_info():
    info = plsc.get_sparse_core_info()
    return info.num_cores, info.num_subcores


def _sc_gather_hist(hist_pidx, item_pidx, itab2, n_hist, n_side):
    nc, ns = _sc_info()
    nw = nc * ns
    lanes = 128
    rows_w = n_hist // nw
    streams_w = rows_w // lanes
    s_per_chunk = 5
    chunks = streams_w // s_per_chunk
    chunk_rows = s_per_chunk * lanes
    side_w = n_side // nw // lanes

    mesh = plsc.VectorSubcoreMesh(core_axis_name="c", subcore_axis_name="s")
    f32 = jnp.float32

    @functools.partial(
        pl.kernel,
        out_type=(
            jax.ShapeDtypeStruct((n_hist, lanes), f32),
            jax.ShapeDtypeStruct((n_side, lanes), f32),
        ),
        mesh=mesh,
        compiler_params=pltpu.CompilerParams(use_tc_tiling_on_sc=True),
        scratch_types=[
            pltpu.VMEM((streams_w, lanes), jnp.int32),
            pltpu.VMEM((chunk_rows, lanes), f32),
            pltpu.VMEM((side_w, lanes), jnp.int32),
            pltpu.VMEM((lanes, lanes), f32),
            pltpu.SemaphoreType.DMA,
        ],
    )
    def gather_kernel(hist_idx_h, item_idx_h, itab_h,
                      hist_out, item_out,
                      idx_v, rows_v, sidx_v, srows_v, sem):
        wid = lax.axis_index("s") * nc + lax.axis_index("c")

        pltpu.sync_copy(item_idx_h.at[wid], sidx_v)
        for j in range(side_w):
            pltpu.async_copy(itab_h.at[sidx_v.at[j]], srows_v, sem).wait()
            pltpu.sync_copy(
                srows_v, item_out.at[pl.ds((wid * side_w + j) * lanes, lanes)])

        pltpu.sync_copy(hist_idx_h.at[wid], idx_v)

        @pl.loop(0, chunks)
        def _chunk(c):
            descs = [
                pltpu.async_copy(
                    itab_h.at[idx_v.at[c * s_per_chunk + jj]],
                    rows_v.at[pl.ds(jj * lanes, lanes)],
                    sem,
                )
                for jj in range(s_per_chunk)
            ]
            for desc in descs:
                desc.wait()
            pltpu.sync_copy(
                rows_v,
                hist_out.at[pl.ds(wid * rows_w + c * chunk_rows, chunk_rows)],
            )

    return gather_kernel(
        hist_pidx.reshape(nw, streams_w, lanes),
        item_pidx.reshape(nw, side_w, lanes),
        itab2)


def _sc_gather_user(user_pidx, utab2, n_side):
    nc, ns = _sc_info()
    nw = nc * ns
    lanes = 128
    side_w = n_side // nw // lanes

    mesh = plsc.VectorSubcoreMesh(core_axis_name="c", subcore_axis_name="s")

    @functools.partial(
        pl.kernel,
        out_type=jax.ShapeDtypeStruct((n_side, lanes), jnp.float32),
        mesh=mesh,
        compiler_params=pltpu.CompilerParams(use_tc_tiling_on_sc=True),
        scratch_types=[
            pltpu.VMEM((side_w, lanes), jnp.int32),
            pltpu.VMEM((lanes, lanes), jnp.float32),
            pltpu.SemaphoreType.DMA,
        ],
    )
    def gather_kernel(user_idx_h, utab_h, user_out, sidx_v, srows_v, sem):
        wid = lax.axis_index("s") * nc + lax.axis_index("c")
        pltpu.sync_copy(user_idx_h.at[wid], sidx_v)
        for j in range(side_w):
            pltpu.async_copy(utab_h.at[sidx_v.at[j]], srows_v, sem).wait()
            pltpu.sync_copy(
                srows_v, user_out.at[pl.ds((wid * side_w + j) * lanes, lanes)])

    return gather_kernel(user_pidx.reshape(nw, side_w, lanes), utab2)


def _tc_dense(hist3, hpar3, user_pk, upar, item_pk, ipar,
              w_int, w1u_t, w1h_t, b1r, w2r, b2r, d):
    n_l, n_b, _ = hist3.shape
    lps = 2 if n_l % 2 == 0 else 1
    f32 = jnp.float32

    def half(packed, par):
        return jnp.where(par == 1, packed[:, d:], packed[:, :d])

    def body(hist_ref, hpar_ref, u_ref, upar_ref, it_ref, ipar_ref,
             wint_ref, w1u_ref, w1h_ref, b1_ref, w2_ref, b2_ref,
             uout_ref, iout_ref, inter_ref, upart_s):
        step = pl.program_id(0)

        @pl.when(step == 0)
        def _init():
            u = half(u_ref[...], upar_ref[...])
            it = half(it_ref[...], ipar_ref[...])
            upart_s[...] = (
                jnp.dot(u, w1u_ref[...], preferred_element_type=f32)
                + b1_ref[...]
            )
            t = jnp.dot(u, wint_ref[...], preferred_element_type=f32)
            inter_ref[...] = jnp.sum(t * it, axis=1, keepdims=True)
            uout_ref[...] = u
            iout_ref[...] = it

        cols = jax.lax.broadcasted_iota(jnp.int32, hpar_ref.shape, 1)
        par = hpar_ref[...]
        hists = []
        for s in range(lps):
            hp = jnp.sum(jnp.where(cols == lps * step + s, par, 0),
                         axis=1, keepdims=True)
            hists.append(half(hist_ref[s], hp))
        hh = jnp.concatenate(hists, axis=0) if lps > 1 else hists[0]
        mm = jnp.dot(hh.astype(jnp.bfloat16),
                     w1h_ref[...].astype(jnp.bfloat16),
                     preferred_element_type=f32)
        up = upart_s[...]
        if lps > 1:
            up = jnp.concatenate([up] * lps, axis=0)
        h = jnp.maximum(mm + up, 0.0)
        a = jax.nn.sigmoid(
            jnp.sum(h * w2_ref[...], axis=1, keepdims=True) + b2_ref[0, 0]
        )
        c = a * hh
        acc = c[:n_b]
        for s in range(1, lps):
            acc = acc + c[s * n_b:(s + 1) * n_b]
        uout_ref[...] += acc

    full = lambda shape: pl.BlockSpec(shape, lambda l: (0,) * len(shape))
    return pl.pallas_call(
        body,
        grid=(n_l // lps,),
        in_specs=[
            pl.BlockSpec((lps, n_b, 2 * d), lambda l: (l, 0, 0)),
            full((n_b, n_l)),
            full((n_b, 2 * d)),
            full((n_b, 1)),
            full((n_b, 2 * d)),
            full((n_b, 1)),
            full((d, d)),
            full((d, d)),
            full((d, d)),
            full((1, d)),
            full((1, d)),
            full((1, 1)),
        ],
        out_specs=[full((n_b, d)), full((n_b, d)), full((n_b, 1))],
        out_shape=[
            jax.ShapeDtypeStruct((n_b, d), f32),
            jax.ShapeDtypeStruct((n_b, d), f32),
            jax.ShapeDtypeStruct((n_b, 1), f32),
        ],
        scratch_shapes=[pltpu.VMEM((n_b, d), f32)],
    )(hist3, hpar3, user_pk, upar, item_pk, ipar,
      w_int, w1u_t, w1h_t, b1r, w2r, b2r)


def kernel(user_ids, item_ids, user_history, user_table, item_table,
           W_int, w1, b1, w2, b2):
    n_b, n_l = user_history.shape
    d = user_table.shape[1]
    h_dim = w1.shape[0]

    itab2, off, tail_start = _tc_pack(item_table, 4096)
    utab2, _, _ = _tc_pack(user_table, 4096)

    def map_ids(i):
        pidx = jnp.where(
            i >= tail_start, i - tail_start + off,
            jnp.where(i >= off, i - off, i))
        hbit = ((i >= off) & (i < tail_start)).astype(jnp.int32)
        return pidx, hbit

    hist_idx = user_history.T.reshape(-1)
    hpidx, _ = map_ids(hist_idx)
    upidx, ubit = map_ids(user_ids)
    ipidx, ibit = map_ids(item_ids)
    hbit_bm = ((user_history >= off)
               & (user_history < tail_start)).astype(jnp.int32)
    hist_rows, item_pk = _sc_gather_hist(
        hpidx, ipidx, itab2, n_l * n_b, n_b)
    user_pk = _sc_gather_user(upidx, utab2, n_b)

    user_out, item_emb, interaction = _tc_dense(
        hist_rows.reshape(n_l, n_b, 2 * d), hbit_bm,
        user_pk, ubit.reshape(n_b, 1),
        item_pk, ibit.reshape(n_b, 1),
        W_int, w1[:, :d].T, w1[:, d:].T,
        b1.reshape(1, h_dim), w2.reshape(1, h_dim),
        b2.reshape(1, 1).astype(jnp.float32), d)

    return (user_out, item_emb, interaction)


# speedup vs baseline: 1.9813x; 1.0828x over previous
import functools

import jax
import jax.numpy as jnp
from jax import lax
from jax.experimental import pallas as pl
from jax.experimental.pallas import tpu as pltpu
from jax.experimental.pallas import tpu_sc as plsc


def _tc_pack(table, n_cols):
    v, d = table.shape
    pairs = v // (2 * n_cols)
    off = pairs * n_cols
    tail_start = 2 * off
    steps = pairs + 1
    f32 = jnp.float32

    tail = jax.lax.slice(table, (tail_start, 0), (v, d))
    tail = jnp.pad(tail, ((0, n_cols - (v - tail_start)), (0, 0)))

    def body(lo_ref, hi_ref, tail_ref, out_ref):
        step = pl.program_id(0)

        @pl.when(step < pairs)
        def _pair():
            out_ref[...] = jnp.concatenate(
                [lo_ref[...].T, hi_ref[...].T], axis=1)

        @pl.when(step == pairs)
        def _tail():
            out_ref[...] = jnp.concatenate(
                [tail_ref[...], jnp.zeros_like(tail_ref[...])], axis=1)

    packed = pl.pallas_call(
        body,
        grid=(steps,),
        in_specs=[
            pl.BlockSpec((d, n_cols), lambda l: (0, jnp.minimum(l, pairs - 1))),
            pl.BlockSpec(
                (d, n_cols),
                lambda l: (0, jnp.minimum(l + pairs, 2 * pairs - 1))),
            pl.BlockSpec((n_cols, d), lambda l: (0, 0)),
        ],
        out_specs=pl.BlockSpec((n_cols, 2 * d), lambda l: (l, 0)),
        out_shape=jax.ShapeDtypeStruct((steps * n_cols, 2 * d), f32),
    )(table.T, table.T, tail)
    return packed, off, tail_start


def _sc_info():
    info = plsc.get_sparse_core_info()
    return info.num_cores, info.num_subcores


def _sc_gather_hist(hist_pidx, item_pidx, itab2, n_hist, n_side):
    nc, ns = _sc_info()
    nw = nc * ns
    lanes = 128
    rows_w = n_hist // nw
    streams_w = rows_w // lanes
    s_per_chunk = 5
    chunks = streams_w // s_per_chunk
    chunk_rows = s_per_chunk * lanes
    side_w = n_side // nw // lanes

    mesh = plsc.VectorSubcoreMesh(core_axis_name="c", subcore_axis_name="s")
    f32 = jnp.float32

    @functools.partial(
        pl.kernel,
        out_type=(
            jax.ShapeDtypeStruct((n_hist, lanes), f32),
            jax.ShapeDtypeStruct((n_side, lanes), f32),
        ),
        mesh=mesh,
        compiler_params=pltpu.CompilerParams(use_tc_tiling_on_sc=True),
        scratch_types=[
            pltpu.VMEM((streams_w, lanes), jnp.int32),
            pltpu.VMEM((chunk_rows, lanes), f32),
            pltpu.VMEM((side_w, lanes), jnp.int32),
            pltpu.VMEM((lanes, lanes), f32),
            pltpu.SemaphoreType.DMA,
        ],
    )
    def gather_kernel(hist_idx_h, item_idx_h, itab_h,
                      hist_out, item_out,
                      idx_v, rows_v, sidx_v, srows_v, sem):
        wid = lax.axis_index("s") * nc + lax.axis_index("c")

        pltpu.sync_copy(item_idx_h.at[wid], sidx_v)
        for j in range(side_w):
            pltpu.async_copy(itab_h.at[sidx_v.at[j]], srows_v, sem).wait()
            pltpu.sync_copy(
                srows_v, item_out.at[pl.ds((wid * side_w + j) * lanes, lanes)])

        pltpu.sync_copy(hist_idx_h.at[wid], idx_v)

        @pl.loop(0, chunks)
        def _chunk(c):
            descs = [
                pltpu.async_copy(
                    itab_h.at[idx_v.at[c * s_per_chunk + jj]],
                    rows_v.at[pl.ds(jj * lanes, lanes)],
                    sem,
                )
                for jj in range(s_per_chunk)
            ]
            for desc in descs:
                desc.wait()
            pltpu.sync_copy(
                rows_v,
                hist_out.at[pl.ds(wid * rows_w + c * chunk_rows, chunk_rows)],
            )

    return gather_kernel(
        hist_pidx.reshape(nw, streams_w, lanes),
        item_pidx.reshape(nw, side_w, lanes),
        itab2)


def _sc_gather_user(user_pidx, utab2, n_side):
    nc, ns = _sc_info()
    nw = nc * ns
    lanes = 128
    side_w = n_side // nw // lanes

    mesh = plsc.VectorSubcoreMesh(core_axis_name="c", subcore_axis_name="s")

    @functools.partial(
        pl.kernel,
        out_type=jax.ShapeDtypeStruct((n_side, lanes), jnp.float32),
        mesh=mesh,
        compiler_params=pltpu.CompilerParams(use_tc_tiling_on_sc=True),
        scratch_types=[
            pltpu.VMEM((side_w, lanes), jnp.int32),
            pltpu.VMEM((lanes, lanes), jnp.float32),
            pltpu.SemaphoreType.DMA,
        ],
    )
    def gather_kernel(user_idx_h, utab_h, user_out, sidx_v, srows_v, sem):
        wid = lax.axis_index("s") * nc + lax.axis_index("c")
        pltpu.sync_copy(user_idx_h.at[wid], sidx_v)
        for j in range(side_w):
            pltpu.async_copy(utab_h.at[sidx_v.at[j]], srows_v, sem).wait()
            pltpu.sync_copy(
                srows_v, user_out.at[pl.ds((wid * side_w + j) * lanes, lanes)])

    return gather_kernel(user_pidx.reshape(nw, side_w, lanes), utab2)


def _tc_dense(hist3, hpar3, user_pk, upar, item_pk, ipar,
              w_int, w1u_t, w1h_t, b1r, w2r, b2r, d):
    n_l, n_b, _ = hist3.shape
    lps = 2 if n_l % 2 == 0 else 1
    f32 = jnp.float32

    def half(packed, par):
        return jnp.where(par == 1, packed[:, d:], packed[:, :d])

    def body(hist_ref, hpar_ref, u_ref, upar_ref, it_ref, ipar_ref,
             wint_ref, w1u_ref, w1h_ref, b1_ref, w2_ref, b2_ref,
             uout_ref, iout_ref, inter_ref, upart_s):
        step = pl.program_id(0)

        @pl.when(step == 0)
        def _init():
            u = half(u_ref[...], upar_ref[...])
            it = half(it_ref[...], ipar_ref[...])
            upart_s[...] = (
                jnp.dot(u, w1u_ref[...], preferred_element_type=f32)
                + b1_ref[...]
            )
            t = jnp.dot(u, wint_ref[...], preferred_element_type=f32)
            inter_ref[...] = jnp.sum(t * it, axis=1, keepdims=True)
            uout_ref[...] = u
            iout_ref[...] = it

        cols = jax.lax.broadcasted_iota(jnp.int32, hpar_ref.shape, 1)
        par = hpar_ref[...]
        hists = []
        for s in range(lps):
            hp = jnp.sum(jnp.where(cols == lps * step + s, par, 0),
                         axis=1, keepdims=True)
            hists.append(half(hist_ref[s], hp))
        hh = jnp.concatenate(hists, axis=0) if lps > 1 else hists[0]
        mm = jnp.dot(hh.astype(jnp.bfloat16),
                     w1h_ref[...].astype(jnp.bfloat16),
                     preferred_element_type=f32)
        up = upart_s[...]
        if lps > 1:
            up = jnp.concatenate([up] * lps, axis=0)
        h = jnp.maximum(mm + up, 0.0)
        a = jax.nn.sigmoid(
            jnp.dot(h.astype(jnp.bfloat16),
                    w2_ref[...].astype(jnp.bfloat16).T,
                    preferred_element_type=f32) + b2_ref[0, 0]
        )
        c = a * hh
        acc = c[:n_b]
        for s in range(1, lps):
            acc = acc + c[s * n_b:(s + 1) * n_b]
        uout_ref[...] += acc

    full = lambda shape: pl.BlockSpec(shape, lambda l: (0,) * len(shape))
    return pl.pallas_call(
        body,
        grid=(n_l // lps,),
        in_specs=[
            pl.BlockSpec((lps, n_b, 2 * d), lambda l: (l, 0, 0)),
            full((n_b, n_l)),
            full((n_b, 2 * d)),
            full((n_b, 1)),
            full((n_b, 2 * d)),
            full((n_b, 1)),
            full((d, d)),
            full((d, d)),
            full((d, d)),
            full((1, d)),
            full((1, d)),
            full((1, 1)),
        ],
        out_specs=[full((n_b, d)), full((n_b, d)), full((n_b, 1))],
        out_shape=[
            jax.ShapeDtypeStruct((n_b, d), f32),
            jax.ShapeDtypeStruct((n_b, d), f32),
            jax.ShapeDtypeStruct((n_b, 1), f32),
        ],
        scratch_shapes=[pltpu.VMEM((n_b, d), f32)],
    )(hist3, hpar3, user_pk, upar, item_pk, ipar,
      w_int, w1u_t, w1h_t, b1r, w2r, b2r)


def kernel(user_ids, item_ids, user_history, user_table, item_table,
           W_int, w1, b1, w2, b2):
    n_b, n_l = user_history.shape
    d = user_table.shape[1]
    h_dim = w1.shape[0]

    itab2, off, tail_start = _tc_pack(item_table, 8192)
    utab2, _, _ = _tc_pack(user_table, 8192)

    def map_ids(i):
        pidx = jnp.where(
            i >= tail_start, i - tail_start + off,
            jnp.where(i >= off, i - off, i))
        hbit = ((i >= off) & (i < tail_start)).astype(jnp.int32)
        return pidx, hbit

    hist_idx = user_history.T.reshape(-1)
    hpidx, _ = map_ids(hist_idx)
    upidx, ubit = map_ids(user_ids)
    ipidx, ibit = map_ids(item_ids)
    hbit_bm = ((user_history >= off)
               & (user_history < tail_start)).astype(jnp.int32)
    hist_rows, item_pk = _sc_gather_hist(
        hpidx, ipidx, itab2, n_l * n_b, n_b)
    user_pk = _sc_gather_user(upidx, utab2, n_b)

    user_out, item_emb, interaction = _tc_dense(
        hist_rows.reshape(n_l, n_b, 2 * d), hbit_bm,
        user_pk, ubit.reshape(n_b, 1),
        item_pk, ibit.reshape(n_b, 1),
        W_int, w1[:, :d].T, w1[:, d:].T,
        b1.reshape(1, h_dim), w2.reshape(1, h_dim),
        b2.reshape(1, 1).astype(jnp.float32), d)

    return (user_out, item_emb, interaction)
